# baseline (speedup 1.0000x reference)
import functools

import jax
import jax.numpy as jnp
from jax import lax
from jax.experimental import pallas as pl
from jax.experimental.pallas import tpu as pltpu
from jax.experimental.pallas import tpu_sc as plsc

_EMB = 128
_SEEDS = 50
_SEEDS_PAD = 56
_NW = 32
_CHUNK_B = 4


def _ranks_body(randt_ref, rankst_ref):
    rt = randt_ref[...]
    lt = rt[None, :, :] < rt[:, None, :]
    rankst_ref[...] = jnp.sum(lt.astype(jnp.int32), axis=1)


def _row_body(row_ref):
    row_ref[...] = jnp.zeros_like(row_ref)


def _sc_col_body(ranks_hbm, col_hbm, buf_a, buf_b, rk, sem_a, sem_b):
    batch_size = col_hbm.shape[0]
    bpw = batch_size // _NW
    nch = bpw // _CHUNK_B
    bufs = (buf_a, buf_b)
    sems = (sem_a, sem_b)
    wid = lax.axis_index("s") * 2 + lax.axis_index("c")
    lane = lax.iota(jnp.int32, 16)
    ones = jnp.full((16,), 1.0, jnp.float32)
    zero = jnp.zeros((16,), jnp.float32)
    for buf in bufs:
        for b in range(_CHUNK_B):
            for r in range(_SEEDS_PAD):
                for k in range(_EMB // 16):
                    buf[b, r, pl.ds(k * 16, 16)] = zero
    pltpu.sync_copy(
        ranks_hbm.at[pl.ds(wid * bpw * _EMB, bpw * _EMB)], rk
    )

    def _pokes(buf, ch, val):
        for lb in range(_CHUNK_B):
            bi = jnp.full((16,), lb, jnp.int32)
            for cc in range(4):
                rvec = rk[pl.ds((ch * _CHUNK_B + lb) * _EMB + cc * 16, 16)]
                c = cc * 16 + lane
                plsc.store_scatter(buf, [bi, rvec, c], val, mask=c < _SEEDS)

    pending = [None, None]
    for ch in range(nch):
        p = ch % 2
        if pending[p] is not None:
            for cp in pending[p]:
                cp.wait()
            _pokes(bufs[p], ch - 2, zero)
        _pokes(bufs[p], ch, ones)
        base_b = wid * bpw + ch * _CHUNK_B
        pending[p] = [
            pltpu.async_copy(
                bufs[p].at[lb, pl.ds(0, _SEEDS), :],
                col_hbm.at[base_b + lb],
                sems[p],
            )
            for lb in range(_CHUNK_B)
        ]
    for p in (0, 1):
        for cp in pending[p]:
            cp.wait()


def kernel(problems):
    batch_size, job_cnt, machine_cnt = problems.shape
    seed_cnt = max(machine_cnt, _SEEDS)
    rand = jax.random.uniform(
        jax.random.key(42), (batch_size, seed_cnt), dtype=jnp.float32
    )
    rand_t = rand.T
    ranks_t = pl.pallas_call(
        _ranks_body,
        out_shape=jax.ShapeDtypeStruct((seed_cnt, batch_size), jnp.int32),
    )(rand_t)
    ranks = jnp.pad(
        ranks_t.T, ((0, 0), (0, _EMB - seed_cnt)), constant_values=127
    )

    sc_col = functools.partial(
        pl.kernel,
        mesh=plsc.VectorSubcoreMesh(core_axis_name="c", subcore_axis_name="s"),
        out_type=jax.ShapeDtypeStruct(
            (batch_size, machine_cnt, _EMB), jnp.float32
        ),
        scratch_types=[
            pltpu.VMEM((_CHUNK_B, _SEEDS_PAD, _EMB), jnp.float32),
            pltpu.VMEM((_CHUNK_B, _SEEDS_PAD, _EMB), jnp.float32),
            pltpu.VMEM((batch_size // _NW * _EMB,), jnp.int32),
            pltpu.SemaphoreType.DMA,
            pltpu.SemaphoreType.DMA,
        ],
        compiler_params=pltpu.CompilerParams(needs_layout_passes=False),
    )(_sc_col_body)
    blk = 64
    row_emb = pl.pallas_call(
        _row_body,
        grid=(batch_size // blk,),
        out_specs=pl.BlockSpec((blk, job_cnt, _EMB), lambda i: (i, 0, 0)),
        out_shape=jax.ShapeDtypeStruct(
            (batch_size, job_cnt, _EMB), jnp.float32
        ),
    )()
    col_emb = sc_col(ranks.reshape(-1))
    return (row_emb, col_emb)


# speedup vs baseline: 1.0054x; 1.0054x over previous
import functools

import jax
import jax.numpy as jnp
from jax import lax
from jax.experimental import pallas as pl
from jax.experimental.pallas import tpu as pltpu
from jax.experimental.pallas import tpu_sc as plsc

_EMB = 128
_SEEDS = 50
_SEEDS_PAD = 56
_NC = 1
_NW = _NC * 16
_CHUNK_B = 4


def _ranks_body(randt_ref, rankst_ref):
    rt = randt_ref[...]
    lt = rt[None, :, :] < rt[:, None, :]
    rankst_ref[...] = jnp.sum(lt.astype(jnp.int32), axis=1)


def _row_body(row_ref):
    row_ref[...] = jnp.zeros_like(row_ref)


def _sc_col_body(ranks_hbm, col_hbm, buf_a, buf_b, rk, sem_a, sem_b):
    batch_size = col_hbm.shape[0]
    bpw = batch_size // _NW
    nch = bpw // _CHUNK_B
    bufs = (buf_a, buf_b)
    sems = (sem_a, sem_b)
    wid = lax.axis_index("s") * _NC + lax.axis_index("c")
    lane = lax.iota(jnp.int32, 16)
    ones = jnp.full((16,), 1.0, jnp.float32)
    zero = jnp.zeros((16,), jnp.float32)
    for buf in bufs:
        for b in range(_CHUNK_B):
            for r in range(_SEEDS_PAD):
                for k in range(_EMB // 16):
                    buf[b, r, pl.ds(k * 16, 16)] = zero
    pltpu.sync_copy(
        ranks_hbm.at[pl.ds(wid * bpw * _EMB, bpw * _EMB)], rk
    )

    def _pokes(buf, ch, val):
        for lb in range(_CHUNK_B):
            bi = jnp.full((16,), lb, jnp.int32)
            for cc in range(4):
                rvec = rk[pl.ds((ch * _CHUNK_B + lb) * _EMB + cc * 16, 16)]
                c = cc * 16 + lane
                plsc.store_scatter(buf, [bi, rvec, c], val, mask=c < _SEEDS)

    pending = [None, None]
    for ch in range(nch):
        p = ch % 2
        if pending[p] is not None:
            for cp in pending[p]:
                cp.wait()
            _pokes(bufs[p], ch - 2, zero)
        _pokes(bufs[p], ch, ones)
        base_b = wid * bpw + ch * _CHUNK_B
        pending[p] = [
            pltpu.async_copy(
                bufs[p].at[lb, pl.ds(0, _SEEDS), :],
                col_hbm.at[base_b + lb],
                sems[p],
            )
            for lb in range(_CHUNK_B)
        ]
    for p in (0, 1):
        for cp in pending[p]:
            cp.wait()


def kernel(problems):
    batch_size, job_cnt, machine_cnt = problems.shape
    seed_cnt = max(machine_cnt, _SEEDS)
    rand = jax.random.uniform(
        jax.random.key(42), (batch_size, seed_cnt), dtype=jnp.float32
    )
    rand_t = rand.T
    ranks_t = pl.pallas_call(
        _ranks_body,
        out_shape=jax.ShapeDtypeStruct((seed_cnt, batch_size), jnp.int32),
    )(rand_t)
    ranks = jnp.pad(
        ranks_t.T, ((0, 0), (0, _EMB - seed_cnt)), constant_values=127
    )

    sc_col = functools.partial(
        pl.kernel,
        mesh=plsc.VectorSubcoreMesh(
            core_axis_name="c", subcore_axis_name="s", num_cores=_NC
        ),
        out_type=jax.ShapeDtypeStruct(
            (batch_size, machine_cnt, _EMB), jnp.float32
        ),
        scratch_types=[
            pltpu.VMEM((_CHUNK_B, _SEEDS_PAD, _EMB), jnp.float32),
            pltpu.VMEM((_CHUNK_B, _SEEDS_PAD, _EMB), jnp.float32),
            pltpu.VMEM((batch_size // _NW * _EMB,), jnp.int32),
            pltpu.SemaphoreType.DMA,
            pltpu.SemaphoreType.DMA,
        ],
        compiler_params=pltpu.CompilerParams(needs_layout_passes=False),
    )(_sc_col_body)
    blk = 64
    row_emb = pl.pallas_call(
        _row_body,
        grid=(batch_size // blk,),
        out_specs=pl.BlockSpec((blk, job_cnt, _EMB), lambda i: (i, 0, 0)),
        out_shape=jax.ShapeDtypeStruct(
            (batch_size, job_cnt, _EMB), jnp.float32
        ),
    )()
    col_emb = sc_col(ranks.reshape(-1))
    return (row_emb, col_emb)
